# Initial kernel scaffold; baseline (speedup 1.0000x reference)
import jax
import jax.numpy as jnp
from jax.experimental import pallas as pl


def kernel(xyz1, xyz2, points1, points2, W0, b0, g0, beta0, W1, b1, g1, beta1):
    raise NotImplementedError("write your pallas kernel here")


# speedup vs baseline: 34.7160x; 34.7160x over previous
import jax
import jax.numpy as jnp
from jax.experimental import pallas as pl


def _knn_interp_body(x1_ref, x2_ref, p1_ref, p2_ref, x_out_ref, ga_ref):
    x1 = x1_ref[0]
    x2 = x2_ref[0]
    p1 = p1_ref[0]
    p2 = p2_ref[0]

    n1 = jnp.sum(x1 * x1, axis=0)
    n2 = jnp.sum(x2 * x2, axis=0)
    cross = jax.lax.dot_general(
        x2, x1, (((0,), (0,)), ((), ())),
        preferred_element_type=jnp.float32)
    d = n2[:, None] + n1[None, :] - 2.0 * cross

    m1 = jnp.min(d, axis=0)
    d1 = jnp.where(d == m1[None, :], jnp.inf, d)
    m2 = jnp.min(d1, axis=0)
    d2 = jnp.where(d1 == m2[None, :], jnp.inf, d1)
    m3 = jnp.min(d2, axis=0)

    mask = d <= m3[None, :]
    recip = jnp.where(mask, 1.0 / (d + 1e-8), 0.0)
    norm = jnp.sum(recip, axis=0)
    h = recip * (1.0 / norm)[None, :]

    interp = jnp.dot(p2, h, preferred_element_type=jnp.float32)
    xt = jnp.concatenate([p1, interp], axis=0)
    x_out_ref[0] = xt

    ones = jnp.ones((8, xt.shape[1]), jnp.float32)
    xa = jnp.concatenate([xt, ones], axis=0)
    g = jax.lax.dot_general(
        xa, xa, (((1,), (1,)), ((), ())),
        preferred_element_type=jnp.float32)

    @pl.when((pl.program_id(0) == 0) & (pl.program_id(1) == 0))
    def _():
        ga_ref[...] = jnp.zeros_like(ga_ref)

    ga_ref[...] += g


def _mlp_gram_body(x_ref, w_ref, y_out_ref, ga_ref):
    xt = x_ref[0]
    ones = jnp.ones((8, xt.shape[1]), jnp.float32)
    xa = jnp.concatenate([xt, ones], axis=0)
    y = jnp.maximum(
        jnp.dot(w_ref[...], xa, preferred_element_type=jnp.float32), 0.0)
    y_out_ref[0] = y
    ya = jnp.concatenate([y, ones], axis=0)
    g = jax.lax.dot_general(
        ya, ya, (((1,), (1,)), ((), ())),
        preferred_element_type=jnp.float32)

    @pl.when((pl.program_id(0) == 0) & (pl.program_id(1) == 0))
    def _():
        ga_ref[...] = jnp.zeros_like(ga_ref)

    ga_ref[...] += g


def _mlp_final_body(x_ref, w_ref, y_out_ref):
    xt = x_ref[0]
    ones = jnp.ones((8, xt.shape[1]), jnp.float32)
    xa = jnp.concatenate([xt, ones], axis=0)
    y_out_ref[0] = jnp.maximum(
        jnp.dot(w_ref[...], xa, preferred_element_type=jnp.float32), 0.0)


def _fold_bn(G, s, M, W, b, g, beta):
    xbar = s / M
    wm = W @ xbar
    mean = wm + b
    q = jnp.sum((W @ G) * W, axis=1) / M
    var = q + 2.0 * b * wm + b * b - mean * mean
    scale = g / jnp.sqrt(var + 1e-5)
    Wp = scale[:, None] * W
    bp = scale * (b - mean) + beta
    C_out = W.shape[0]
    return jnp.concatenate(
        [Wp, bp[:, None], jnp.zeros((C_out, 7), jnp.float32)], axis=1)


@jax.jit
def kernel(xyz1, xyz2, points1, points2, W0, b0, g0, beta0, W1, b1, g1, beta1):
    B, _, N = xyz1.shape
    S = xyz2.shape[2]
    Dp = points1.shape[1]
    C = 2 * Dp
    Ca = C + 8
    M = B * N

    Nt = min(256, N)
    grid_a = (B, N // Nt)

    x_full, ga0 = pl.pallas_call(
        _knn_interp_body,
        grid=grid_a,
        in_specs=[
            pl.BlockSpec((1, 3, Nt), lambda b, n: (b, 0, n)),
            pl.BlockSpec((1, 3, S), lambda b, n: (b, 0, 0)),
            pl.BlockSpec((1, Dp, Nt), lambda b, n: (b, 0, n)),
            pl.BlockSpec((1, Dp, S), lambda b, n: (b, 0, 0)),
        ],
        out_specs=[
            pl.BlockSpec((1, C, Nt), lambda b, n: (b, 0, n)),
            pl.BlockSpec((Ca, Ca), lambda b, n: (0, 0)),
        ],
        out_shape=[
            jax.ShapeDtypeStruct((B, C, N), jnp.float32),
            jax.ShapeDtypeStruct((Ca, Ca), jnp.float32),
        ],
    )(xyz1, xyz2, points1, points2)

    W0a = _fold_bn(ga0[:C, :C], ga0[C, :C], M, W0, b0, g0, beta0)

    Nt2 = min(512, N)
    grid_m = (B, N // Nt2)

    x2_full, ga1 = pl.pallas_call(
        _mlp_gram_body,
        grid=grid_m,
        in_specs=[
            pl.BlockSpec((1, C, Nt2), lambda b, n: (b, 0, n)),
            pl.BlockSpec((128, Ca), lambda b, n: (0, 0)),
        ],
        out_specs=[
            pl.BlockSpec((1, 128, Nt2), lambda b, n: (b, 0, n)),
            pl.BlockSpec((136, 136), lambda b, n: (0, 0)),
        ],
        out_shape=[
            jax.ShapeDtypeStruct((B, 128, N), jnp.float32),
            jax.ShapeDtypeStruct((136, 136), jnp.float32),
        ],
    )(x_full, W0a)

    W1a = _fold_bn(ga1[:128, :128], ga1[128, :128], M, W1, b1, g1, beta1)

    out = pl.pallas_call(
        _mlp_final_body,
        grid=grid_m,
        in_specs=[
            pl.BlockSpec((1, 128, Nt2), lambda b, n: (b, 0, n)),
            pl.BlockSpec((128, 136), lambda b, n: (0, 0)),
        ],
        out_specs=pl.BlockSpec((1, 128, Nt2), lambda b, n: (b, 0, n)),
        out_shape=jax.ShapeDtypeStruct((B, 128, N), jnp.float32),
    )(x2_full, W1a)

    return out


# speedup vs baseline: 49.4449x; 1.4243x over previous
import jax
import jax.numpy as jnp
from jax.experimental import pallas as pl
from jax.experimental.pallas import tpu as pltpu


def _merge3(a1, b1, c1, a2, b2, c2):
    s1 = jnp.minimum(a1, a2)
    t = jnp.maximum(a1, a2)
    u = jnp.minimum(b1, b2)
    v = jnp.maximum(b1, b2)
    s2 = jnp.minimum(t, u)
    s3 = jnp.minimum(jnp.minimum(jnp.maximum(t, u), v), jnp.minimum(c1, c2))
    return s1, s2, s3


def _top3_small(d, chunk=128):
    S = d.shape[0]
    big = jnp.full((chunk, d.shape[1]), jnp.inf, jnp.float32)
    a, b, c = big, big, big
    for i in range(S // chunk):
        x = d[i * chunk:(i + 1) * chunk]
        na = jnp.minimum(a, x)
        t = jnp.maximum(a, x)
        nb = jnp.minimum(b, t)
        t2 = jnp.maximum(b, t)
        c = jnp.minimum(c, t2)
        a, b = na, nb
    rows = chunk
    while rows > 1:
        h = rows // 2
        a, b, c = _merge3(a[:h], b[:h], c[:h], a[h:], b[h:], c[h:])
        rows = h
    return a, b, c


def _make_knn_body(B, ntiles, S):
    def body(x1_ref, x2_ref, p1_ref, p2_ref, x_out_ref, ga_ref):
        bi = pl.program_id(0)
        ni = pl.program_id(1)

        x1 = x1_ref[0]
        x2 = x2_ref[0]
        Nt = x1.shape[1]
        n1 = jnp.sum(x1 * x1, axis=0)
        n2 = jnp.sum(x2 * x2, axis=0)
        cross = jax.lax.dot_general(
            x2, x1, (((0,), (0,)), ((), ())),
            preferred_element_type=jnp.float32)
        d = (n2[:, None] + n1[None, :]) - 2.0 * cross

        m1, m2, m3 = _top3_small(d)
        r1 = 1.0 / (m1 + 1e-8)
        r2 = 1.0 / (m2 + 1e-8)
        r3 = 1.0 / (m3 + 1e-8)
        invn = 1.0 / (r1 + r2 + r3)
        hu = jnp.where(d <= m3, 1.0 / (d + 1e-8), 0.0)
        interp = jnp.dot(p2_ref[0], hu,
                         preferred_element_type=jnp.float32) * invn

        xt = jnp.concatenate([p1_ref[0], interp], axis=0)
        x_out_ref[0] = xt
        ones = jnp.ones((8, Nt), jnp.float32)
        xa = jnp.concatenate([xt, ones], axis=0)
        gacc = jax.lax.dot_general(
            xa, xa, (((1,), (1,)), ((), ())),
            preferred_element_type=jnp.float32)

        @pl.when((bi == 0) & (ni == 0))
        def _():
            ga_ref[...] = jnp.zeros_like(ga_ref)

        ga_ref[...] += gacc

    return body


def _mlp_gram_body(x_ref, w_ref, y_out_ref, ga_ref):
    xt = x_ref[0]
    ones = jnp.ones((8, xt.shape[1]), jnp.float32)
    xa = jnp.concatenate([xt, ones], axis=0)
    y = jnp.maximum(
        jnp.dot(w_ref[...], xa, preferred_element_type=jnp.float32), 0.0)
    y_out_ref[0] = y
    ya = jnp.concatenate([y, ones], axis=0)
    g = jax.lax.dot_general(
        ya, ya, (((1,), (1,)), ((), ())),
        preferred_element_type=jnp.float32)

    @pl.when((pl.program_id(0) == 0) & (pl.program_id(1) == 0))
    def _():
        ga_ref[...] = jnp.zeros_like(ga_ref)

    ga_ref[...] += g


def _mlp_final_body(x_ref, w_ref, y_out_ref):
    xt = x_ref[0]
    ones = jnp.ones((8, xt.shape[1]), jnp.float32)
    xa = jnp.concatenate([xt, ones], axis=0)
    y_out_ref[0] = jnp.maximum(
        jnp.dot(w_ref[...], xa, preferred_element_type=jnp.float32), 0.0)


def _fold_bn(G, s, M, W, b, g, beta):
    xbar = s / M
    wm = W @ xbar
    mean = wm + b
    q = jnp.sum((W @ G) * W, axis=1) / M
    var = q + 2.0 * b * wm + b * b - mean * mean
    scale = g / jnp.sqrt(var + 1e-5)
    Wp = scale[:, None] * W
    bp = scale * (b - mean) + beta
    C_out = W.shape[0]
    return jnp.concatenate(
        [Wp, bp[:, None], jnp.zeros((C_out, 7), jnp.float32)], axis=1)


@jax.jit
def kernel(xyz1, xyz2, points1, points2, W0, b0, g0, beta0, W1, b1, g1, beta1):
    B, _, N = xyz1.shape
    S = xyz2.shape[2]
    Dp = points1.shape[1]
    C = 2 * Dp
    Ca = C + 8
    M = B * N

    Nt = min(512, N)
    nta = N // Nt

    x_full, ga0 = pl.pallas_call(
        _make_knn_body(B, nta, S),
        grid=(B, nta),
        in_specs=[
            pl.BlockSpec((1, 3, Nt), lambda b, n: (b, 0, n)),
            pl.BlockSpec((1, 3, S), lambda b, n: (b, 0, 0)),
            pl.BlockSpec((1, Dp, Nt), lambda b, n: (b, 0, n)),
            pl.BlockSpec((1, Dp, S), lambda b, n: (b, 0, 0)),
        ],
        out_specs=[
            pl.BlockSpec((1, C, Nt), lambda b, n: (b, 0, n)),
            pl.BlockSpec((Ca, Ca), lambda b, n: (0, 0)),
        ],
        out_shape=[
            jax.ShapeDtypeStruct((B, C, N), jnp.float32),
            jax.ShapeDtypeStruct((Ca, Ca), jnp.float32),
        ],
    )(xyz1, xyz2, points1, points2)

    W0a = _fold_bn(ga0[:C, :C], ga0[C, :C], M, W0, b0, g0, beta0)

    Nt2 = min(512, N)
    grid_m = (B, N // Nt2)

    x2_full, ga1 = pl.pallas_call(
        _mlp_gram_body,
        grid=grid_m,
        in_specs=[
            pl.BlockSpec((1, C, Nt2), lambda b, n: (b, 0, n)),
            pl.BlockSpec((128, Ca), lambda b, n: (0, 0)),
        ],
        out_specs=[
            pl.BlockSpec((1, 128, Nt2), lambda b, n: (b, 0, n)),
            pl.BlockSpec((136, 136), lambda b, n: (0, 0)),
        ],
        out_shape=[
            jax.ShapeDtypeStruct((B, 128, N), jnp.float32),
            jax.ShapeDtypeStruct((136, 136), jnp.float32),
        ],
    )(x_full, W0a)

    W1a = _fold_bn(ga1[:128, :128], ga1[128, :128], M, W1, b1, g1, beta1)

    out = pl.pallas_call(
        _mlp_final_body,
        grid=grid_m,
        in_specs=[
            pl.BlockSpec((1, 128, Nt2), lambda b, n: (b, 0, n)),
            pl.BlockSpec((128, 136), lambda b, n: (0, 0)),
        ],
        out_specs=pl.BlockSpec((1, 128, Nt2), lambda b, n: (b, 0, n)),
        out_shape=jax.ShapeDtypeStruct((B, 128, N), jnp.float32),
    )(x2_full, W1a)

    return out
